# baseline (speedup 1.0000x reference)
import functools

import jax
import jax.numpy as jnp
from jax.experimental import pallas as pl
from jax.experimental.pallas import tpu as pltpu

_NUM_EXPERTS = 64
_TOP_K = 8
_DIM = 4096
_T = 512


def _router_kernel(x_ref, wt_ref, bias_ref, ts_ref, idx_ref, cnt_ref):
    b = pl.program_id(0)
    t = pl.program_id(1)

    logits = jnp.dot(
        x_ref[0],
        wt_ref[...],
        preferred_element_type=jnp.float32,
        precision=jax.lax.Precision.DEFAULT,
    )

    m = jnp.max(logits, axis=1, keepdims=True)
    e = jnp.exp(logits - m)
    p = e / jnp.sum(e, axis=1, keepdims=True)

    work = p + bias_ref[...]
    iota = jax.lax.broadcasted_iota(jnp.int32, (_T, _NUM_EXPERTS), 1)

    vals = []
    idxs = []
    for _ in range(_TOP_K):
        mx = jnp.max(work, axis=1, keepdims=True)
        cand = jnp.where(work == mx, iota, _NUM_EXPERTS)
        sel = jnp.min(cand, axis=1, keepdims=True)
        onehot = iota == sel
        vals.append(jnp.sum(jnp.where(onehot, p, 0.0), axis=1, keepdims=True))
        idxs.append(sel)
        work = jnp.where(onehot, -jnp.inf, work)

    counts = jnp.sum((work == -jnp.inf).astype(jnp.int32), axis=0, keepdims=True)

    ts_ref[0] = jnp.concatenate(vals, axis=1)
    idx_ref[0] = jnp.concatenate(idxs, axis=1)

    @pl.when(jnp.logical_and(b == 0, t == 0))
    def _init():
        cnt_ref[...] = jnp.zeros_like(cnt_ref)

    cnt_ref[pl.ds(b, 1), :] += counts


@functools.partial(jax.jit, static_argnames=())
def _router(x, expert_bias, wt):
    B, S, D = x.shape
    grid = (B, S // _T)
    return pl.pallas_call(
        _router_kernel,
        grid=grid,
        in_specs=[
            pl.BlockSpec((1, _T, D), lambda b, t: (b, t, 0)),
            pl.BlockSpec((D, _NUM_EXPERTS), lambda b, t: (0, 0)),
            pl.BlockSpec((1, _NUM_EXPERTS), lambda b, t: (0, 0)),
        ],
        out_specs=[
            pl.BlockSpec((1, _T, _TOP_K), lambda b, t: (b, t, 0)),
            pl.BlockSpec((1, _T, _TOP_K), lambda b, t: (b, t, 0)),
            pl.BlockSpec((B, _NUM_EXPERTS), lambda b, t: (0, 0)),
        ],
        out_shape=[
            jax.ShapeDtypeStruct((B, S, _TOP_K), jnp.float32),
            jax.ShapeDtypeStruct((B, S, _TOP_K), jnp.int32),
            jax.ShapeDtypeStruct((B, _NUM_EXPERTS), jnp.int32),
        ],
        compiler_params=pltpu.CompilerParams(
            dimension_semantics=("arbitrary", "arbitrary"),
        ),
    )(x, wt, expert_bias)


def kernel(x, expert_bias, W):
    top_scores, idx, counts = _router(
        x, expert_bias.reshape(1, _NUM_EXPERTS), W.T
    )
    return (top_scores, idx, counts)


# speedup vs baseline: 1.0011x; 1.0011x over previous
import functools

import jax
import jax.numpy as jnp
from jax.experimental import pallas as pl
from jax.experimental.pallas import tpu as pltpu

_NUM_EXPERTS = 64
_TOP_K = 8
_DIM = 4096
_T = 512


_C = 64


def _router_kernel(x_ref, wt_ref, bias_ref, ts_ref, idx_ref, cnt_ref):
    b = pl.program_id(0)
    t = pl.program_id(1)

    iota = jax.lax.broadcasted_iota(jnp.int32, (_C, _NUM_EXPERTS), 1)
    counts = jnp.zeros((1, _NUM_EXPERTS), dtype=jnp.int32)
    for c in range(_T // _C):
        sl = slice(c * _C, (c + 1) * _C)
        logits = jnp.dot(
            x_ref[0, sl, :],
            wt_ref[...],
            preferred_element_type=jnp.float32,
            precision=jax.lax.Precision.DEFAULT,
        )

        m = jnp.max(logits, axis=1, keepdims=True)
        e = jnp.exp(logits - m)
        p = e / jnp.sum(e, axis=1, keepdims=True)

        work = p + bias_ref[...]

        vals = []
        idxs = []
        for _ in range(_TOP_K):
            mx = jnp.max(work, axis=1, keepdims=True)
            cand = jnp.where(work == mx, iota, _NUM_EXPERTS)
            sel = jnp.min(cand, axis=1, keepdims=True)
            onehot = iota == sel
            vals.append(jnp.sum(jnp.where(onehot, p, 0.0), axis=1, keepdims=True))
            idxs.append(sel)
            work = jnp.where(onehot, -jnp.inf, work)

        counts += jnp.sum((work == -jnp.inf).astype(jnp.int32), axis=0, keepdims=True)

        ts_ref[0, sl, :] = jnp.concatenate(vals, axis=1)
        idx_ref[0, sl, :] = jnp.concatenate(idxs, axis=1)

    @pl.when(jnp.logical_and(b == 0, t == 0))
    def _init():
        cnt_ref[...] = jnp.zeros_like(cnt_ref)

    cnt_ref[pl.ds(b, 1), :] += counts


@functools.partial(jax.jit, static_argnames=())
def _router(x, expert_bias, wt):
    B, S, D = x.shape
    grid = (B, S // _T)
    return pl.pallas_call(
        _router_kernel,
        grid=grid,
        in_specs=[
            pl.BlockSpec((1, _T, D), lambda b, t: (b, t, 0)),
            pl.BlockSpec((D, _NUM_EXPERTS), lambda b, t: (0, 0)),
            pl.BlockSpec((1, _NUM_EXPERTS), lambda b, t: (0, 0)),
        ],
        out_specs=[
            pl.BlockSpec((1, _T, _TOP_K), lambda b, t: (b, t, 0)),
            pl.BlockSpec((1, _T, _TOP_K), lambda b, t: (b, t, 0)),
            pl.BlockSpec((B, _NUM_EXPERTS), lambda b, t: (0, 0)),
        ],
        out_shape=[
            jax.ShapeDtypeStruct((B, S, _TOP_K), jnp.float32),
            jax.ShapeDtypeStruct((B, S, _TOP_K), jnp.int32),
            jax.ShapeDtypeStruct((B, _NUM_EXPERTS), jnp.int32),
        ],
        compiler_params=pltpu.CompilerParams(
            dimension_semantics=("arbitrary", "arbitrary"),
        ),
    )(x, wt, expert_bias)


def kernel(x, expert_bias, W):
    top_scores, idx, counts = _router(
        x, expert_bias.reshape(1, _NUM_EXPERTS), W.T
    )
    return (top_scores, idx, counts)


# speedup vs baseline: 1.2138x; 1.2125x over previous
import functools

import jax
import jax.numpy as jnp
from jax.experimental import pallas as pl
from jax.experimental.pallas import tpu as pltpu

_NUM_EXPERTS = 64
_TOP_K = 8
_DIM = 4096
_T = 512


_C = 64


def _router_kernel(x_ref, wt_ref, bias_ref, ts_ref, idx_ref, cnt_ref):
    b = pl.program_id(0)
    t = pl.program_id(1)

    iota = jax.lax.broadcasted_iota(jnp.int32, (_C, _NUM_EXPERTS), 1)
    counts = jnp.zeros((1, _NUM_EXPERTS), dtype=jnp.int32)
    for c in range(_T // _C):
        sl = slice(c * _C, (c + 1) * _C)
        logits = jnp.dot(
            x_ref[0, sl, :],
            wt_ref[...],
            preferred_element_type=jnp.float32,
            precision=jax.lax.Precision.DEFAULT,
        )

        m = jnp.max(logits, axis=1, keepdims=True)
        e = jnp.exp(logits - m)
        p = e / jnp.sum(e, axis=1, keepdims=True)

        work = p + bias_ref[...]

        vals = []
        idxs = []
        for _ in range(_TOP_K):
            sel = jnp.argmax(work, axis=1, keepdims=True)
            onehot = iota == sel
            vals.append(jnp.sum(jnp.where(onehot, p, 0.0), axis=1, keepdims=True))
            idxs.append(sel)
            work = jnp.where(onehot, -jnp.inf, work)

        counts += jnp.sum((work == -jnp.inf).astype(jnp.int32), axis=0, keepdims=True)

        ts_ref[0, sl, :] = jnp.concatenate(vals, axis=1)
        idx_ref[0, sl, :] = jnp.concatenate(idxs, axis=1)

    @pl.when(jnp.logical_and(b == 0, t == 0))
    def _init():
        cnt_ref[...] = jnp.zeros_like(cnt_ref)

    cnt_ref[pl.ds(b, 1), :] += counts


@functools.partial(jax.jit, static_argnames=())
def _router(x, expert_bias, wt):
    B, S, D = x.shape
    grid = (B, S // _T)
    return pl.pallas_call(
        _router_kernel,
        grid=grid,
        in_specs=[
            pl.BlockSpec((1, _T, D), lambda b, t: (b, t, 0)),
            pl.BlockSpec((D, _NUM_EXPERTS), lambda b, t: (0, 0)),
            pl.BlockSpec((1, _NUM_EXPERTS), lambda b, t: (0, 0)),
        ],
        out_specs=[
            pl.BlockSpec((1, _T, _TOP_K), lambda b, t: (b, t, 0)),
            pl.BlockSpec((1, _T, _TOP_K), lambda b, t: (b, t, 0)),
            pl.BlockSpec((B, _NUM_EXPERTS), lambda b, t: (0, 0)),
        ],
        out_shape=[
            jax.ShapeDtypeStruct((B, S, _TOP_K), jnp.float32),
            jax.ShapeDtypeStruct((B, S, _TOP_K), jnp.int32),
            jax.ShapeDtypeStruct((B, _NUM_EXPERTS), jnp.int32),
        ],
        compiler_params=pltpu.CompilerParams(
            dimension_semantics=("arbitrary", "arbitrary"),
        ),
    )(x, wt, expert_bias)


def kernel(x, expert_bias, W):
    top_scores, idx, counts = _router(
        x, expert_bias.reshape(1, _NUM_EXPERTS), W.T
    )
    return (top_scores, idx, counts)


# speedup vs baseline: 1.3737x; 1.1317x over previous
import functools

import jax
import jax.numpy as jnp
from jax.experimental import pallas as pl
from jax.experimental.pallas import tpu as pltpu

_NUM_EXPERTS = 64
_TOP_K = 8
_DIM = 4096
_T = 1024


_C = 64


def _router_kernel(x_ref, wt_ref, bias_ref, ts_ref, idx_ref, cnt_ref):
    b = pl.program_id(0)
    t = pl.program_id(1)

    iota = jax.lax.broadcasted_iota(jnp.int32, (_C, _NUM_EXPERTS), 1)
    counts = jnp.zeros((1, _NUM_EXPERTS), dtype=jnp.int32)
    for c in range(_T // _C):
        sl = slice(c * _C, (c + 1) * _C)
        logits = jnp.dot(
            x_ref[0, sl, :],
            wt_ref[...],
            preferred_element_type=jnp.float32,
            precision=jax.lax.Precision.DEFAULT,
        )

        m = jnp.max(logits, axis=1, keepdims=True)
        e = jnp.exp(logits - m)
        p = e / jnp.sum(e, axis=1, keepdims=True)

        work = p + bias_ref[...]

        vals = []
        idxs = []
        for _ in range(_TOP_K):
            sel = jnp.argmax(work, axis=1, keepdims=True)
            onehot = iota == sel
            vals.append(jnp.sum(jnp.where(onehot, p, 0.0), axis=1, keepdims=True))
            idxs.append(sel)
            work = jnp.where(onehot, -jnp.inf, work)

        counts += jnp.sum((work == -jnp.inf).astype(jnp.int32), axis=0, keepdims=True)

        ts_ref[0, sl, :] = jnp.concatenate(vals, axis=1)
        idx_ref[0, sl, :] = jnp.concatenate(idxs, axis=1)

    @pl.when(jnp.logical_and(b == 0, t == 0))
    def _init():
        cnt_ref[...] = jnp.zeros_like(cnt_ref)

    cnt_ref[pl.ds(b, 1), :] += counts


@functools.partial(jax.jit, static_argnames=())
def _router(x, expert_bias, wt):
    B, S, D = x.shape
    grid = (B, S // _T)
    return pl.pallas_call(
        _router_kernel,
        grid=grid,
        in_specs=[
            pl.BlockSpec((1, _T, D), lambda b, t: (b, t, 0)),
            pl.BlockSpec((D, _NUM_EXPERTS), lambda b, t: (0, 0)),
            pl.BlockSpec((1, _NUM_EXPERTS), lambda b, t: (0, 0)),
        ],
        out_specs=[
            pl.BlockSpec((1, _T, _TOP_K), lambda b, t: (b, t, 0)),
            pl.BlockSpec((1, _T, _TOP_K), lambda b, t: (b, t, 0)),
            pl.BlockSpec((B, _NUM_EXPERTS), lambda b, t: (0, 0)),
        ],
        out_shape=[
            jax.ShapeDtypeStruct((B, S, _TOP_K), jnp.float32),
            jax.ShapeDtypeStruct((B, S, _TOP_K), jnp.int32),
            jax.ShapeDtypeStruct((B, _NUM_EXPERTS), jnp.int32),
        ],
        compiler_params=pltpu.CompilerParams(
            dimension_semantics=("arbitrary", "arbitrary"),
        ),
    )(x, wt, expert_bias)


def kernel(x, expert_bias, W):
    top_scores, idx, counts = _router(
        x, expert_bias.reshape(1, _NUM_EXPERTS), W.T
    )
    return (top_scores, idx, counts)
